# baseline (speedup 1.0000x reference)
import jax
import jax.numpy as jnp
from jax import lax
from jax.experimental import pallas as pl
from jax.experimental.pallas import tpu as pltpu
from jax.experimental.pallas import tpu_sc as plsc

B = 8
N = 20000
NCLS = 80
MAX_NMS = 2048
MAX_DET = 300
CONF_THRES = 0.25
IOU_THRES = 0.45
BASE_BITS = 0x3E800000

BA = 5000
LANES = 16
KPAD = 384
CPAD = MAX_NMS + 4 * LANES
NLOG = MAX_DET * NCLS


def _lpad_kernel(log_ref, lpad_ref):
    lpad_ref[0, :, 0:NCLS] = log_ref[0]


def _lpad(logits):
    return pl.pallas_call(
        _lpad_kernel,
        grid=(B, N // BA),
        in_specs=[pl.BlockSpec((1, BA, NCLS), lambda b, a: (b, a, 0))],
        out_specs=pl.BlockSpec((1, BA, 128), lambda b, a: (b, a, 0)),
        out_shape=jax.ShapeDtypeStruct((B, N, 128), jnp.float32),
    )(logits)


def _score_kernel(pred_ref, aux_ref):
    p = pred_ref[0]
    obj = p[:, 4]
    prod = p[:, 5:85] * obj[:, None]
    conf = jnp.max(prod, axis=1)
    iot = lax.broadcasted_iota(jnp.int32, (p.shape[0], NCLS), 1)
    j = jnp.min(jnp.where(prod == conf[:, None], iot, NCLS), axis=1)
    valid = (obj > CONF_THRES) & (conf > CONF_THRES)
    score = jnp.where(valid, conf, -1.0)
    aux_ref[0, 0, 0] = p[:, 0] - p[:, 2] / 2.0
    aux_ref[0, 0, 1] = p[:, 1] - p[:, 3] / 2.0
    aux_ref[0, 0, 2] = p[:, 0] + p[:, 2] / 2.0
    aux_ref[0, 0, 3] = p[:, 1] + p[:, 3] / 2.0
    aux_ref[0, 0, 4] = j.astype(jnp.float32)
    aux_ref[0, 0, 5] = score


def _phase1(prediction):
    return pl.pallas_call(
        _score_kernel,
        grid=(B, N // BA),
        in_specs=[pl.BlockSpec((1, BA, 85), lambda b, a: (b, a, 0))],
        out_specs=pl.BlockSpec((1, 1, 6, BA), lambda b, a: (b, a, 0, 0)),
        out_shape=jax.ShapeDtypeStruct((B, N // BA, 6, BA), jnp.float32),
    )(prediction)


def _splat_i(v):
    return jnp.full((LANES,), v, jnp.int32)


def _nms_body(aux_hbm, log_hbm, det_hbm, ol_hbm,
              S, H, CS, CX1, CY1, CX2, CY2, CCl, CI, GIB, KI, OD, OLB, sem):
    nc = 2
    wid = lax.axis_index("s") * nc + lax.axis_index("c")

    @pl.when(wid < B)
    def _():
        b = wid
        lanes = lax.iota(jnp.int32, LANES)
        zf = jnp.zeros((LANES,), jnp.float32)
        zi = jnp.zeros((LANES,), jnp.int32)
        onesi = jnp.ones((LANES,), jnp.int32)
        negf = jnp.full((LANES,), -1.0, jnp.float32)

        for ablk in range(N // BA):
            pltpu.sync_copy(
                aux_hbm.at[pl.ds(b * 6 * N + ablk * 6 * BA + 5 * BA, BA)],
                S.at[pl.ds(ablk * BA, BA)])

        def _z16(i, _):
            OD[pl.ds(i * LANES, LANES)] = zf
            return 0
        lax.fori_loop(0, (KPAD * 16) // LANES, _z16, 0)

        def _zki(i, _):
            KI[pl.ds(i * LANES, LANES)] = zi
            return 0
        lax.fori_loop(0, KPAD // LANES, _zki, 0)

        def _radix_pass(shift, prefix, prefmask, need, count_valid):
            def _zh(i, _):
                H[pl.ds(i * LANES, LANES)] = zi
                return 0
            lax.fori_loop(0, 256, _zh, 0)

            def _hist(i, cnt):
                v = S[pl.ds(i * LANES, LANES)]
                bits = plsc.bitcast(v, jnp.uint32) - BASE_BITS
                pos = v > 0.0
                m = pos & ((bits & prefmask) == prefix)
                bk = ((bits >> shift) & 0xFF).astype(jnp.int32)
                plsc.addupdate_scatter(H, [bk * LANES + lanes], onesi, mask=m)
                if count_valid:
                    cnt = cnt + jnp.where(pos, 1, 0)
                return cnt
            cntv = lax.fori_loop(0, N // LANES, _hist, zi)

            def _cond(st):
                bk, cum, found = st
                return (found == 0) & (bk >= 0)

            def _body(st):
                bk, cum, found = st
                s = jnp.sum(H[pl.ds(bk * LANES, LANES)])
                hit = (cum + s) >= need
                return (jnp.where(hit, bk, bk - 1),
                        jnp.where(hit, cum, cum + s),
                        jnp.where(hit, 1, 0))
            bk, cum, _f = lax.while_loop(_cond, _body, (255, 0, 0))
            bk = jnp.maximum(bk, 0)
            return bk.astype(jnp.uint32), need - cum, jnp.sum(cntv)

        b0, need0, nvalid = _radix_pass(jnp.uint32(16), jnp.uint32(0),
                                        jnp.uint32(0), MAX_NMS, True)

        def _select(_):
            b1, need1, _c = _radix_pass(jnp.uint32(8), b0 << 16,
                                        jnp.uint32(0x00FF0000), need0, False)
            b2, need2, _c = _radix_pass(jnp.uint32(0),
                                        (b0 << 16) | (b1 << 8),
                                        jnp.uint32(0x00FFFF00), need1, False)
            tau = jnp.uint32(BASE_BITS) + ((b0 << 16) | (b1 << 8) | b2)
            return tau, need2

        def _no_select(_):
            return jnp.uint32(0), 0

        tau, m_need = lax.cond(nvalid > MAX_NMS, _select, _no_select, 0)

        def _initc(i, _):
            CS[pl.ds(i * LANES, LANES)] = negf
            CI[pl.ds(i * LANES, LANES)] = zi
            return 0
        lax.fori_loop(0, CPAD // LANES, _initc, 0)

        def _compact(i, st):
            n, ties = st
            v = S[pl.ds(i * LANES, LANES)]
            bits = plsc.bitcast(v, jnp.uint32)
            pos = v > 0.0
            eqm = pos & (bits == tau)
            eqi = jnp.where(eqm, 1, 0)
            pref_exc = plsc.cumsum(eqi) - eqi
            take = eqm & ((ties + pref_exc) < m_need)
            elig = (pos & (bits > tau)) | take
            plsc.store_compressed(CS.at[pl.ds(n, LANES)], v, mask=elig)
            plsc.store_compressed(CI.at[pl.ds(n, LANES)],
                                  lanes + i * LANES, mask=elig)
            return (n + jnp.sum(jnp.where(elig, 1, 0)), ties + jnp.sum(eqi))
        lax.fori_loop(0, N // LANES, _compact, (0, 0))

        def _gi(i, _):
            ci = CI[pl.ds(i * LANES, LANES)]
            base = b * (6 * N) + (ci // BA) * (6 * BA) + (ci % BA)
            for c in range(5):
                GIB[pl.ds(c * CPAD + i * LANES, LANES)] = base + c * BA
            return 0
        lax.fori_loop(0, CPAD // LANES, _gi, 0)

        planes = [CX1, CY1, CX2, CY2, CCl]
        descs = []
        for c in range(5):
            for j in range(MAX_NMS // 128):
                descs.append(pltpu.async_copy(
                    aux_hbm.at[GIB.at[pl.ds(c * CPAD + 128 * j, 128)]],
                    planes[c].at[pl.ds(128 * j, 128)], sem))
        for d in descs:
            d.wait()

        def _amax(i, st):
            m, ri = st
            v = CS[pl.ds(i * LANES, LANES)]
            upd = v > m
            return jnp.maximum(v, m), jnp.where(upd, i, ri)

        def _argmax(nv):
            m, ri = lax.fori_loop(0, nv, _amax,
                                  (jnp.full((LANES,), -2.0, jnp.float32), zi))
            best = jnp.max(m)
            gidx = jnp.where(m == best, ri * LANES + lanes, N)
            return jnp.min(gidx), best

        w0, best0 = _argmax(MAX_NMS // LANES)

        def _gcond(st):
            k, w, best, nv = st
            return (k < MAX_DET) & (best > 0.0)

        def _gbody(st):
            k, w, best, nv = st

            do_c = (k == 16) | (k == 48) | (k == 96) | (k == 160)

            def _do_compact(args):
                w_in, best_in, nv_in = args

                def _cp(i, n):
                    sl_i = pl.ds(i * LANES, LANES)
                    v = CS[sl_i]
                    m = v > 0.0
                    sl = pl.ds(n, LANES)
                    plsc.store_compressed(CS.at[sl], v, mask=m)
                    plsc.store_compressed(CX1.at[sl], CX1[sl_i], mask=m)
                    plsc.store_compressed(CY1.at[sl], CY1[sl_i], mask=m)
                    plsc.store_compressed(CX2.at[sl], CX2[sl_i], mask=m)
                    plsc.store_compressed(CY2.at[sl], CY2[sl_i], mask=m)
                    plsc.store_compressed(CCl.at[sl], CCl[sl_i], mask=m)
                    plsc.store_compressed(CI.at[sl], CI[sl_i], mask=m)
                    return n + jnp.sum(jnp.where(m, 1, 0))
                n2 = lax.fori_loop(0, nv_in, _cp, 0)
                nv4 = ((n2 + 4 * LANES - 1) // (4 * LANES)) * 4

                @pl.when(n2 > 0)
                def _():
                    for t in range(4):
                        q = nv4 - 1 - t
                        tail = pl.ds(q * LANES, LANES)
                        tv = CS[tail]
                        CS[tail] = jnp.where(q * LANES + lanes >= n2,
                                             -1.0, tv)
                w2, best2 = _argmax(nv4)
                return w2, best2, nv4

            w, best, nv = lax.cond(do_c, _do_compact,
                                   lambda a: a, (w, best, nv))

            ws = _splat_i(w)
            vwx1 = plsc.load_gather(CX1, [ws])
            vwy1 = plsc.load_gather(CY1, [ws])
            vwx2 = plsc.load_gather(CX2, [ws])
            vwy2 = plsc.load_gather(CY2, [ws])
            vwcl = plsc.load_gather(CCl, [ws])
            vwi = plsc.load_gather(CI, [ws])
            varea = (vwx2 - vwx1) * (vwy2 - vwy1)
            vbest = zf + best

            dvec = jnp.where(
                lanes == 0, vwx1,
                jnp.where(lanes == 1, vwy1,
                          jnp.where(lanes == 2, vwx2,
                                    jnp.where(lanes == 3, vwy2,
                                              jnp.where(lanes == 4, vbest,
                                                        vwcl)))))
            plsc.store_scatter(OD, [_splat_i(k * 16) + lanes], dvec,
                               mask=lanes < 6)
            plsc.store_scatter(KI, [_splat_i(k)], vwi + b * N,
                               mask=lanes == 0)

            def _sup1(i, st2):
                m, ri = st2
                sl = pl.ds(i * LANES, LANES)
                s = CS[sl]
                x1 = CX1[sl]
                y1 = CY1[sl]
                x2 = CX2[sl]
                y2 = CY2[sl]
                ltx = jnp.maximum(vwx1, x1)
                lty = jnp.maximum(vwy1, y1)
                rbx = jnp.minimum(vwx2, x2)
                rby = jnp.minimum(vwy2, y2)
                iw = jnp.maximum(rbx - ltx, 0.0)
                ih = jnp.maximum(rby - lty, 0.0)
                inter = iw * ih
                area = (x2 - x1) * (y2 - y1)
                denom = ((varea + area) - inter) + 1e-9
                iou = inter / denom
                s2 = jnp.where(iou > IOU_THRES, -1.0, s)
                CS[sl] = s2
                upd = s2 > m
                return jnp.maximum(s2, m), jnp.where(upd, i, ri)

            def _sup4(i4, st2):
                for u in range(4):
                    st2 = _sup1(i4 * 4 + u, st2)
                return st2

            m, ri = lax.fori_loop(
                0, nv // 4, _sup4,
                (jnp.full((LANES,), -2.0, jnp.float32), zi))
            nbest = jnp.max(m)
            gidx = jnp.where(m == nbest, ri * LANES + lanes, N)
            nw = jnp.min(gidx)
            return k + 1, nw, nbest, nv

        kfin, _w, _b2, _nv = lax.while_loop(
            _gcond, _gbody, (0, w0, best0, MAX_NMS // LANES))

        def _lg(kk, _):
            r = jnp.max(plsc.load_gather(KI, [_splat_i(kk)]))
            pltpu.async_copy(log_hbm.at[pl.ds(r * 128, NCLS)],
                             OLB.at[pl.ds(kk * NCLS, NCLS)], sem)
            return 0
        lax.fori_loop(0, MAX_DET, _lg, 0)
        pltpu.make_async_copy(ol_hbm.at[b], OLB, sem).wait()

        def _zl(i, _):
            OLB[pl.ds(i * LANES, LANES)] = zf
            return 0
        lax.fori_loop(kfin * (NCLS // LANES), NLOG // LANES, _zl, 0)

        pltpu.sync_copy(OD, det_hbm.at[b])
        pltpu.sync_copy(OLB, ol_hbm.at[b])


def _phase2(auxf, logf):
    mesh = plsc.VectorSubcoreMesh(core_axis_name="c", subcore_axis_name="s",
                                  num_cores=2, num_subcores=16)
    f = pl.kernel(
        _nms_body,
        out_type=(
            jax.ShapeDtypeStruct((B, KPAD * 16), jnp.float32),
            jax.ShapeDtypeStruct((B, NLOG), jnp.float32),
        ),
        mesh=mesh,
        compiler_params=pltpu.CompilerParams(needs_layout_passes=False,
                                             use_tc_tiling_on_sc=False),
        scratch_types=[
            pltpu.VMEM((N,), jnp.float32),
            pltpu.VMEM((256 * LANES,), jnp.int32),
            pltpu.VMEM((CPAD,), jnp.float32),
            pltpu.VMEM((CPAD,), jnp.float32),
            pltpu.VMEM((CPAD,), jnp.float32),
            pltpu.VMEM((CPAD,), jnp.float32),
            pltpu.VMEM((CPAD,), jnp.float32),
            pltpu.VMEM((CPAD,), jnp.float32),
            pltpu.VMEM((CPAD,), jnp.int32),
            pltpu.VMEM((5 * CPAD,), jnp.int32),
            pltpu.VMEM((KPAD,), jnp.int32),
            pltpu.VMEM((KPAD * 16,), jnp.float32),
            pltpu.VMEM((NLOG,), jnp.float32),
            pltpu.SemaphoreType.DMA,
        ],
    )
    return f(auxf, logf)


def kernel(prediction, logits):
    aux = _phase1(prediction)
    lpad = _lpad(logits)
    auxf = aux.reshape(B * 6 * N)
    logf = lpad.reshape(B * N * 128)
    det, logp = _phase2(auxf, logf)
    det = det.reshape(B, KPAD, 16)
    logp = logp.reshape(B, MAX_DET, NCLS)
    return jnp.concatenate(
        [det[:, :MAX_DET, :6], logp], axis=-1)


# speedup vs baseline: 1.1264x; 1.1264x over previous
import functools

import jax
import jax.numpy as jnp
from jax import lax
from jax.experimental import pallas as pl
from jax.experimental.pallas import tpu as pltpu
from jax.experimental.pallas import tpu_sc as plsc

B = 8
N = 20000
NCLS = 80
MAX_NMS = 2048
MAX_DET = 300
CONF_THRES = 0.25
IOU_THRES = 0.45
BASE_BITS = 0x3E800000

BA = 2000
LANES = 16
KPAD = 384


def _score_kernel(pred_ref, score_ref, aux_ref):
    p = pred_ref[0]
    obj = p[:, 4]
    prod = p[:, 5:85] * obj[:, None]
    conf = jnp.max(prod, axis=1)
    iot = lax.broadcasted_iota(jnp.int32, (p.shape[0], NCLS), 1)
    j = jnp.min(jnp.where(prod == conf[:, None], iot, NCLS), axis=1)
    valid = (obj > CONF_THRES) & (conf > CONF_THRES)
    score = jnp.where(valid, conf, -1.0)
    score_ref[0, 0] = score
    x1 = p[:, 0] - p[:, 2] / 2.0
    y1 = p[:, 1] - p[:, 3] / 2.0
    x2 = p[:, 0] + p[:, 2] / 2.0
    y2 = p[:, 1] + p[:, 3] / 2.0
    zero = jnp.zeros_like(score)
    aux_ref[0] = jnp.stack(
        [x1, y1, x2, y2, j.astype(jnp.float32), score, zero, zero], axis=1)


def _phase1(prediction):
    scores3, aux = pl.pallas_call(
        _score_kernel,
        grid=(B, N // BA),
        in_specs=[pl.BlockSpec((1, BA, 85), lambda b, a: (b, a, 0))],
        out_specs=[
            pl.BlockSpec((1, 1, BA), lambda b, a: (b * (N // BA) + a, 0, 0)),
            pl.BlockSpec((1, BA, 8), lambda b, a: (b, a, 0)),
        ],
        out_shape=[
            jax.ShapeDtypeStruct((B * (N // BA), 1, BA), jnp.float32),
            jax.ShapeDtypeStruct((B, N, 8), jnp.float32),
        ],
    )(prediction)
    return scores3.reshape(B, N), aux


def _splat_i(v):
    return jnp.full((LANES,), v, jnp.int32)


def _splat_f(v):
    return jnp.full((LANES,), v, jnp.float32)


def _sload_f(ref, idx):
    return jnp.max(plsc.load_gather(ref, [_splat_i(idx)]))


def _sload_i(ref, idx):
    return jnp.max(plsc.load_gather(ref, [_splat_i(idx)]))


def _sstore(ref, idx, val, dtype):
    lanes = lax.iota(jnp.int32, LANES)
    plsc.store_scatter(ref, [_splat_i(idx)],
                       jnp.full((LANES,), val, dtype), mask=lanes == 0)


def _nms_body(score_hbm, aux_hbm, logits_hbm, det_hbm, log_hbm,
              S, H, CS, CX1, CY1, CX2, CY2, CCl, CI, GI, CR, KI, OD, OL, sem):
    nc = 2
    wid = lax.axis_index("s") * nc + lax.axis_index("c")

    @pl.when(wid < B)
    def _():
        b = wid
        lanes = lax.iota(jnp.int32, LANES)
        zf = jnp.zeros((LANES,), jnp.float32)
        zi = jnp.zeros((LANES,), jnp.int32)
        onesi = jnp.ones((LANES,), jnp.int32)
        negf = jnp.full((LANES,), -1.0, jnp.float32)

        pltpu.sync_copy(score_hbm.at[b], S)

        def _z16(i, _):
            OD[pl.ds(i * LANES, LANES)] = zf
            return 0
        lax.fori_loop(0, (KPAD * 16) // LANES, _z16, 0)

        def _zki(i, _):
            KI[pl.ds(i * LANES, LANES)] = zi
            return 0
        lax.fori_loop(0, KPAD // LANES, _zki, 0)

        def _cnt(i, acc):
            v = S[pl.ds(i * LANES, LANES)]
            return acc + jnp.where(v > 0.0, 1, 0)
        nvalid = jnp.sum(lax.fori_loop(0, N // LANES, _cnt, zi))

        def _radix_pass(shift, prefix, prefmask, need):
            def _zh(i, _):
                H[pl.ds(i * LANES, LANES)] = zi
                return 0
            lax.fori_loop(0, 256, _zh, 0)

            def _hist(i, _):
                v = S[pl.ds(i * LANES, LANES)]
                bits = plsc.bitcast(v, jnp.uint32) - BASE_BITS
                m = (v > 0.0) & ((bits & prefmask) == prefix)
                bk = ((bits >> shift) & 0xFF).astype(jnp.int32)
                plsc.addupdate_scatter(H, [bk * LANES + lanes], onesi, mask=m)
                return 0
            lax.fori_loop(0, N // LANES, _hist, 0)

            def _cond(st):
                bk, cum, found = st
                return (found == 0) & (bk >= 0)

            def _body(st):
                bk, cum, found = st
                s = jnp.sum(H[pl.ds(bk * LANES, LANES)])
                hit = (cum + s) >= need
                return (jnp.where(hit, bk, bk - 1),
                        jnp.where(hit, cum, cum + s),
                        jnp.where(hit, 1, 0))
            bk, cum, _f = lax.while_loop(_cond, _body, (255, 0, 0))
            bk = jnp.maximum(bk, 0)
            return bk.astype(jnp.uint32), need - cum

        def _select(_):
            b0, need0 = _radix_pass(jnp.uint32(16), jnp.uint32(0),
                                    jnp.uint32(0), MAX_NMS)
            b1, need1 = _radix_pass(jnp.uint32(8), b0 << 16,
                                    jnp.uint32(0x00FF0000), need0)
            b2, need2 = _radix_pass(jnp.uint32(0),
                                    (b0 << 16) | (b1 << 8),
                                    jnp.uint32(0x00FFFF00), need1)
            tau = jnp.uint32(BASE_BITS) + ((b0 << 16) | (b1 << 8) | b2)
            return tau, need2

        def _no_select(_):
            return jnp.uint32(0), 0

        tau, m_need = lax.cond(nvalid > MAX_NMS, _select, _no_select, 0)

        def _initc(i, _):
            CS[pl.ds(i * LANES, LANES)] = negf
            CI[pl.ds(i * LANES, LANES)] = zi
            return 0
        lax.fori_loop(0, (MAX_NMS + LANES) // LANES, _initc, 0)

        def _compact(i, st):
            n, ties = st
            v = S[pl.ds(i * LANES, LANES)]
            bits = plsc.bitcast(v, jnp.uint32)
            pos = v > 0.0
            eqm = pos & (bits == tau)
            eqi = jnp.where(eqm, 1, 0)
            pref_exc = plsc.cumsum(eqi) - eqi
            take = eqm & ((ties + pref_exc) < m_need)
            elig = (pos & (bits > tau)) | take
            plsc.store_compressed(CS.at[pl.ds(n, LANES)], v, mask=elig)
            plsc.store_compressed(CI.at[pl.ds(n, LANES)],
                                  lanes + i * LANES, mask=elig)
            return (n + jnp.sum(jnp.where(elig, 1, 0)), ties + jnp.sum(eqi))
        ncand, _t = lax.fori_loop(0, N // LANES, _compact, (0, 0))

        def _gi(i, _):
            GI[pl.ds(i * LANES, LANES)] = CI[pl.ds(i * LANES, LANES)] + b * N
            return 0
        lax.fori_loop(0, (MAX_NMS + LANES) // LANES, _gi, 0)

        descs = []
        for j in range(MAX_NMS // 128):
            descs.append(pltpu.async_copy(
                aux_hbm.at[GI.at[pl.ds(128 * j, 128)]],
                CR.at[pl.ds(128 * j, 128), :], sem))
        for d in descs:
            d.wait()

        def _tr(i, _):
            ridx = lanes + i * LANES
            CX1[pl.ds(i * LANES, LANES)] = plsc.load_gather(
                CR, [ridx, _splat_i(0)])
            CY1[pl.ds(i * LANES, LANES)] = plsc.load_gather(
                CR, [ridx, _splat_i(1)])
            CX2[pl.ds(i * LANES, LANES)] = plsc.load_gather(
                CR, [ridx, _splat_i(2)])
            CY2[pl.ds(i * LANES, LANES)] = plsc.load_gather(
                CR, [ridx, _splat_i(3)])
            CCl[pl.ds(i * LANES, LANES)] = plsc.load_gather(
                CR, [ridx, _splat_i(4)])
            return 0
        lax.fori_loop(0, MAX_NMS // LANES, _tr, 0)

        def _amax(i, st):
            m, ri = st
            v = CS[pl.ds(i * LANES, LANES)]
            upd = v > m
            return jnp.maximum(v, m), jnp.where(upd, i, ri)

        def _argmax(nv):
            m, ri = lax.fori_loop(0, nv, _amax,
                                  (jnp.full((LANES,), -2.0, jnp.float32), zi))
            best = jnp.max(m)
            gidx = jnp.where(m == best, ri * LANES + lanes, N)
            return jnp.min(gidx), best

        w0, best0 = _argmax(MAX_NMS // LANES)

        def _gcond(st):
            k, w, best, nv = st
            return (k < MAX_DET) & (best > 0.0)

        def _gbody(st):
            k, w, best, nv = st

            do_c = (k == 16) | (k == 48) | (k == 96) | (k == 160)

            def _do_compact(args):
                w_in, best_in, nv_in = args

                def _cp(i, n):
                    v = CS[pl.ds(i * LANES, LANES)]
                    m = v > 0.0
                    sl = pl.ds(n, LANES)
                    plsc.store_compressed(CS.at[sl], v, mask=m)
                    plsc.store_compressed(
                        CX1.at[sl], CX1[pl.ds(i * LANES, LANES)], mask=m)
                    plsc.store_compressed(
                        CY1.at[sl], CY1[pl.ds(i * LANES, LANES)], mask=m)
                    plsc.store_compressed(
                        CX2.at[sl], CX2[pl.ds(i * LANES, LANES)], mask=m)
                    plsc.store_compressed(
                        CY2.at[sl], CY2[pl.ds(i * LANES, LANES)], mask=m)
                    plsc.store_compressed(
                        CCl.at[sl], CCl[pl.ds(i * LANES, LANES)], mask=m)
                    plsc.store_compressed(
                        CI.at[sl], CI[pl.ds(i * LANES, LANES)], mask=m)
                    return n + jnp.sum(jnp.where(m, 1, 0))
                n2 = lax.fori_loop(0, nv_in, _cp, 0)
                nv2 = (n2 + LANES - 1) // LANES

                @pl.when(n2 > 0)
                def _():
                    tail = pl.ds((nv2 - 1) * LANES, LANES)
                    tv = CS[tail]
                    CS[tail] = jnp.where(
                        (nv2 - 1) * LANES + lanes >= n2, -1.0, tv)
                w2, best2 = _argmax(nv2)
                return w2, best2, nv2

            w, best, nv = lax.cond(do_c, _do_compact,
                                   lambda a: a, (w, best, nv))

            wx1 = _sload_f(CX1, w)
            wy1 = _sload_f(CY1, w)
            wx2 = _sload_f(CX2, w)
            wy2 = _sload_f(CY2, w)
            wcl = _sload_f(CCl, w)
            wid_a = _sload_i(CI, w)
            warea = (wx2 - wx1) * (wy2 - wy1)

            base16 = k * 16
            _sstore(OD, base16 + 0, wx1, jnp.float32)
            _sstore(OD, base16 + 1, wy1, jnp.float32)
            _sstore(OD, base16 + 2, wx2, jnp.float32)
            _sstore(OD, base16 + 3, wy2, jnp.float32)
            _sstore(OD, base16 + 4, best, jnp.float32)
            _sstore(OD, base16 + 5, wcl, jnp.float32)
            _sstore(KI, k, wid_a + b * N, jnp.int32)

            vwx1 = _splat_f(0.0) + wx1
            vwy1 = _splat_f(0.0) + wy1
            vwx2 = _splat_f(0.0) + wx2
            vwy2 = _splat_f(0.0) + wy2
            varea = _splat_f(0.0) + warea

            def _sup(i, st2):
                m, ri = st2
                sl = pl.ds(i * LANES, LANES)
                s = CS[sl]
                x1 = CX1[sl]
                y1 = CY1[sl]
                x2 = CX2[sl]
                y2 = CY2[sl]
                ltx = jnp.maximum(vwx1, x1)
                lty = jnp.maximum(vwy1, y1)
                rbx = jnp.minimum(vwx2, x2)
                rby = jnp.minimum(vwy2, y2)
                iw = jnp.maximum(rbx - ltx, 0.0)
                ih = jnp.maximum(rby - lty, 0.0)
                inter = iw * ih
                area = (x2 - x1) * (y2 - y1)
                denom = ((varea + area) - inter) + 1e-9
                iou = inter / denom
                s2 = jnp.where(iou > IOU_THRES, -1.0, s)
                CS[sl] = s2
                upd = s2 > m
                return jnp.maximum(s2, m), jnp.where(upd, i, ri)

            m, ri = lax.fori_loop(
                0, nv, _sup, (jnp.full((LANES,), -2.0, jnp.float32), zi))
            nbest = jnp.max(m)
            gidx = jnp.where(m == nbest, ri * LANES + lanes, N)
            nw = jnp.min(gidx)
            return k + 1, nw, nbest, nv

        kfin, _w, _b, _nv = lax.while_loop(
            _gcond, _gbody, (0, w0, best0, MAX_NMS // LANES))

        ldescs = []
        for j in range(KPAD // 128):
            ldescs.append(pltpu.async_copy(
                logits_hbm.at[KI.at[pl.ds(128 * j, 128)]],
                OL.at[pl.ds(128 * j, 128), :], sem))
        for d in ldescs:
            d.wait()

        def _zrow(i, _):
            for c in range(NCLS // LANES):
                OL[i, pl.ds(c * LANES, LANES)] = zf
            return 0
        lax.fori_loop(kfin, KPAD, _zrow, 0)

        pltpu.sync_copy(OD, det_hbm.at[b])
        pltpu.sync_copy(OL, log_hbm.at[b])


def _phase2(scores, auxr, logits2d):
    mesh = plsc.VectorSubcoreMesh(core_axis_name="c", subcore_axis_name="s",
                                  num_cores=2, num_subcores=16)
    cp = pltpu.CompilerParams(needs_layout_passes=False,
                              use_tc_tiling_on_sc=False)
    f = pl.kernel(
        _nms_body,
        out_type=(
            jax.ShapeDtypeStruct((B, KPAD * 16), jnp.float32),
            jax.ShapeDtypeStruct((B, KPAD, NCLS), jnp.float32),
        ),
        mesh=mesh,
        compiler_params=cp,
        scratch_types=[
            pltpu.VMEM((N,), jnp.float32),
            pltpu.VMEM((256 * LANES,), jnp.int32),
            pltpu.VMEM((MAX_NMS + LANES,), jnp.float32),
            pltpu.VMEM((MAX_NMS + LANES,), jnp.float32),
            pltpu.VMEM((MAX_NMS + LANES,), jnp.float32),
            pltpu.VMEM((MAX_NMS + LANES,), jnp.float32),
            pltpu.VMEM((MAX_NMS + LANES,), jnp.float32),
            pltpu.VMEM((MAX_NMS + LANES,), jnp.float32),
            pltpu.VMEM((MAX_NMS + LANES,), jnp.int32),
            pltpu.VMEM((MAX_NMS + LANES,), jnp.int32),
            pltpu.VMEM((MAX_NMS, 8), jnp.float32),
            pltpu.VMEM((KPAD,), jnp.int32),
            pltpu.VMEM((KPAD * 16,), jnp.float32),
            pltpu.VMEM((KPAD, NCLS), jnp.float32),
            pltpu.SemaphoreType.DMA,
        ],
    )
    return f(scores, auxr, logits2d)


def kernel(prediction, logits):
    scores, aux = _phase1(prediction)
    auxr = aux.reshape(B * N, 8)
    logits2d = logits.reshape(B * N, NCLS)
    det, logp = _phase2(scores, auxr, logits2d)
    det = det.reshape(B, KPAD, 16)
    return jnp.concatenate(
        [det[:, :MAX_DET, :6], logp[:, :MAX_DET, :]], axis=-1)
